# baseline (speedup 1.0000x reference)
import jax
import jax.numpy as jnp
from jax import lax
from jax.experimental import pallas as pl
from jax.experimental.pallas import tpu as pltpu
from jax.experimental.pallas import tpu_sc as plsc

B = 128
N = 32768
K = 12
L = 16
NW = 32
ROWS_PER_W = B // NW
BLOCK = 1024
SUB = 128
NEG = -3.0e38


def _lane(x, k):
    i = lax.iota(jnp.int32, L)
    return jnp.max(jnp.where(i == k, x, NEG))


def _tree_max(vs):
    while len(vs) > 1:
        vs = [jnp.maximum(vs[i], vs[i + 1]) for i in range(0, len(vs) - 1, 2)] \
            + ([vs[-1]] if len(vs) % 2 else [])
    return vs[0]


def _merge16(av, ai, bv, bi):
    rbv = lax.rev(bv, (0,))
    rbi = lax.rev(bi, (0,))
    take = (av > rbv) | ((av == rbv) & (ai < rbi))
    nv = jnp.where(take, av, rbv)
    ni = jnp.where(take, ai, rbi)
    return plsc.sort_key_val(nv, ni, descending=True)


def _sub_top16(log_v, sbase):
    iota = lax.iota(jnp.int32, L)
    pairs = []
    for u in range(8):
        v = log_v[pl.ds(sbase + u * L, L)]
        idx = sbase + u * L + iota
        pairs.append(plsc.sort_key_val(v, idx, descending=True))
    while len(pairs) > 1:
        pairs = [_merge16(*pairs[i], *pairs[i + 1])
                 for i in range(0, len(pairs), 2)]
    return pairs[0]


def _tie_fixup(cand_v, cand_i, fv_ref, fi_ref, iota):
    for phase in range(2):
        if phase == 0:
            partner = jnp.bitwise_xor(iota, 1)
        else:
            partner = jnp.where((iota >= 1) & (iota <= 14),
                                jnp.bitwise_xor(iota - 1, 1) + 1, iota)
        fv_ref[...] = cand_v
        fi_ref[...] = cand_i
        pv = plsc.load_gather(fv_ref, [partner])
        pi = plsc.load_gather(fi_ref, [partner])
        win = (cand_v > pv) | ((cand_v == pv) & (cand_i < pi))
        lower = iota < partner
        keep = (lower & win) | (~lower & ~win)
        cand_v = jnp.where(keep, cand_v, pv)
        cand_i = jnp.where(keep, cand_i, pi)
    return cand_v, cand_i


def _sc_body(logits_hbm, tgt2_hbm, tab_hbm, out_hbm,
             log_a, log_b, tab_v, res_v, fv_ref, fi_ref, gat_v,
             sem_l, sem_g, sem_o):
    wid = lax.axis_index("c") * 16 + lax.axis_index("s")
    iota = lax.iota(jnp.int32, L)
    pltpu.sync_copy(tab_hbm, tab_v)
    inv_ranks = tab_v[...]
    mask12 = (iota < K).astype(jnp.float32)

    r0 = wid * ROWS_PER_W
    logbufs = [log_a, log_b]
    h_log = pltpu.async_copy(logits_hbm.at[r0], log_a, sem_l)
    out_handles = []

    for k in range(ROWS_PER_W):
        row = r0 + k
        log_v = logbufs[k % 2]
        h_log.wait()
        if k + 1 < ROWS_PER_W:
            h_log = pltpu.async_copy(
                logits_hbm.at[row + 1], logbufs[(k + 1) % 2], sem_l)

        def blk_body(b, carry, log_v=log_v):
            cand_v, cand_i, t = carry
            base = b * BLOCK
            accs = [log_v[pl.ds(base + u * L, L)] for u in range(8)]
            for j in range(8, BLOCK // L):
                accs[j % 8] = jnp.maximum(
                    accs[j % 8], log_v[pl.ds(base + j * L, L)])
            bmax = jnp.max(_tree_max(accs))

            def slow(carry):
                def sb_body(sb, carry):
                    sbase = base + sb * SUB
                    cs = [log_v[pl.ds(sbase + u * L, L)] for u in range(8)]
                    sbmax = jnp.max(_tree_max(cs))

                    def sb_slow(carry):
                        cv, ci, _ = carry
                        sv, si = _sub_top16(log_v, sbase)
                        cv2, ci2 = _merge16(sv, si, cv, ci)
                        return cv2, ci2, _lane(cv2, K - 1)

                    tt = carry[2]
                    return lax.cond(sbmax > tt, lambda: sb_slow(carry),
                                    lambda: carry)
                return lax.fori_loop(0, BLOCK // SUB, sb_body, carry)

            return lax.cond(bmax > t, lambda: slow(carry), lambda: carry)

        cand_v, cand_i, _ = lax.fori_loop(
            0, N // BLOCK, blk_body,
            (jnp.full((L,), NEG, jnp.float32), jnp.zeros((L,), jnp.int32),
             jnp.float32(NEG)))

        cand_v, cand_i = _tie_fixup(cand_v, cand_i, fv_ref, fi_ref, iota)
        grow = row * (N // 128) + lax.shift_right_logical(cand_i, 7)
        pltpu.async_copy(tgt2_hbm.at[grow], gat_v, sem_g).wait()
        bits = plsc.load_gather(
            gat_v, [iota, jnp.bitwise_and(cand_i, 127)]).astype(jnp.float32)
        hits = bits * mask12
        tp = plsc.cumsum(hits)
        metric = jnp.sum(tp * inv_ranks * hits)

        res_v[k, :] = jnp.where(iota == 0, metric, 0.0)
        out_handles.append(
            pltpu.async_copy(res_v.at[k], out_hbm.at[row], sem_o))

    for h in out_handles:
        h.wait()


def _tc_rowsum_body(t_ref, o_ref):
    o_ref[...] = jnp.sum(t_ref[...].astype(jnp.float32), axis=1)


@jax.jit
def _sc_map(logits, targets):
    mesh = plsc.VectorSubcoreMesh(core_axis_name="c", subcore_axis_name="s")
    f = pl.kernel(
        _sc_body,
        out_type=jax.ShapeDtypeStruct((B, L), jnp.float32),
        mesh=mesh,
        scratch_types=[
            pltpu.VMEM((N,), jnp.float32),
            pltpu.VMEM((N,), jnp.float32),
            pltpu.VMEM((L,), jnp.float32),
            pltpu.VMEM((ROWS_PER_W, L), jnp.float32),
            pltpu.VMEM((L,), jnp.float32),
            pltpu.VMEM((L,), jnp.int32),
            pltpu.VMEM((L, 128), jnp.int32),
            pltpu.SemaphoreType.DMA,
            pltpu.SemaphoreType.DMA,
            pltpu.SemaphoreType.DMA,
        ],
        compiler_params=pltpu.CompilerParams(needs_layout_passes=False),
    )
    inv_ranks = 1.0 / (jnp.arange(L, dtype=jnp.float32) + 1.0)
    tgt2 = targets.reshape(B * N // 128, 128)
    metric = f(logits, tgt2, inv_ranks)[:, 0]

    sums = pl.pallas_call(
        _tc_rowsum_body,
        out_shape=jax.ShapeDtypeStruct((B,), jnp.float32),
    )(targets)

    denom = jnp.minimum(jnp.float32(K), sums)
    return jnp.sum(metric / denom)


def kernel(logits, targets):
    return _sc_map(logits, targets)


# speedup vs baseline: 1.4591x; 1.4591x over previous
import jax
import jax.numpy as jnp
from jax import lax
from jax.experimental import pallas as pl
from jax.experimental.pallas import tpu as pltpu
from jax.experimental.pallas import tpu_sc as plsc

B = 128
N = 32768
K = 12
L = 16
NW = 32
ROWS_PER_W = B // NW
BLOCK = 1024
SUB = 128
NEG = -3.0e38


def _lane(x, k):
    i = lax.iota(jnp.int32, L)
    return jnp.max(jnp.where(i == k, x, NEG))


def _tree_max(vs):
    while len(vs) > 1:
        vs = [jnp.maximum(vs[i], vs[i + 1]) for i in range(0, len(vs) - 1, 2)] \
            + ([vs[-1]] if len(vs) % 2 else [])
    return vs[0]


def _merge16(av, ai, bv, bi):
    rbv = lax.rev(bv, (0,))
    rbi = lax.rev(bi, (0,))
    take = (av > rbv) | ((av == rbv) & (ai < rbi))
    nv = jnp.where(take, av, rbv)
    ni = jnp.where(take, ai, rbi)
    return plsc.sort_key_val(nv, ni, descending=True)


def _sub_top16(log_v, sbase):
    iota = lax.iota(jnp.int32, L)
    pairs = []
    for u in range(8):
        v = log_v[pl.ds(sbase + u * L, L)]
        idx = sbase + u * L + iota
        pairs.append(plsc.sort_key_val(v, idx, descending=True))
    while len(pairs) > 1:
        pairs = [_merge16(*pairs[i], *pairs[i + 1])
                 for i in range(0, len(pairs), 2)]
    return pairs[0]


def _tie_fixup(cand_v, cand_i, fv_ref, fi_ref, iota):
    for phase in range(2):
        if phase == 0:
            partner = jnp.bitwise_xor(iota, 1)
        else:
            partner = jnp.where((iota >= 1) & (iota <= 14),
                                jnp.bitwise_xor(iota - 1, 1) + 1, iota)
        fv_ref[...] = cand_v
        fi_ref[...] = cand_i
        pv = plsc.load_gather(fv_ref, [partner])
        pi = plsc.load_gather(fi_ref, [partner])
        win = (cand_v > pv) | ((cand_v == pv) & (cand_i < pi))
        lower = iota < partner
        keep = (lower & win) | (~lower & ~win)
        cand_v = jnp.where(keep, cand_v, pv)
        cand_i = jnp.where(keep, cand_i, pi)
    return cand_v, cand_i


def _sc_body(logits_hbm, targets_hbm, tab_hbm, out_hbm,
             log_a, log_b, tgt_v, tab_v, res_v, fv_ref, fi_ref,
             sem_l, sem_t, sem_o):
    wid = lax.axis_index("c") * 16 + lax.axis_index("s")
    iota = lax.iota(jnp.int32, L)
    pltpu.sync_copy(tab_hbm, tab_v)
    inv_ranks = tab_v[0]
    rec_denom = tab_v[1]
    mask12 = (iota < K).astype(jnp.float32)

    r0 = wid * ROWS_PER_W
    logbufs = [log_a, log_b]
    h_log = pltpu.async_copy(logits_hbm.at[r0], log_a, sem_l)
    h_tgt = pltpu.async_copy(targets_hbm.at[r0], tgt_v, sem_t)
    out_handles = []

    for k in range(ROWS_PER_W):
        row = r0 + k
        log_v = logbufs[k % 2]
        h_log.wait()
        if k + 1 < ROWS_PER_W:
            h_log = pltpu.async_copy(
                logits_hbm.at[row + 1], logbufs[(k + 1) % 2], sem_l)

        def blk_body(b, carry, log_v=log_v):
            cand_v, cand_i, t = carry
            base = b * BLOCK
            accs = []
            for u in range(8):
                a = log_v[pl.ds(base + u * SUB, L)]
                for j in range(1, SUB // L):
                    a = jnp.maximum(a, log_v[pl.ds(base + u * SUB + j * L, L)])
                accs.append(a)
            bmax = jnp.max(_tree_max(accs))

            def slow(carry):
                cv, ci, tt = carry
                tvec = jnp.full((L,), tt)
                bits = jnp.zeros((L,), jnp.int32)
                for u in range(8):
                    cnt = plsc.all_reduce_population_count(accs[u] > tvec)
                    bits = bits | (jnp.minimum(cnt, 1) << u)
                bmap = jnp.max(bits)

                def sb_body(sb, carry):
                    def go(carry):
                        cv, ci, t_in = carry
                        sv, si = _sub_top16(log_v, base + sb * SUB)
                        cv2, ci2 = _merge16(sv, si, cv, ci)
                        return cv2, ci2, t_in
                    hit = (lax.shift_right_logical(bmap, sb) & 1) > 0
                    return lax.cond(hit, lambda: go(carry), lambda: carry)

                cv, ci, _ = lax.fori_loop(0, BLOCK // SUB, sb_body,
                                          (cv, ci, tt))
                return cv, ci, _lane(cv, K - 1)

            return lax.cond(bmax > t, lambda: slow(carry), lambda: carry)

        cand_v, cand_i, _ = lax.fori_loop(
            0, N // BLOCK, blk_body,
            (jnp.full((L,), NEG, jnp.float32), jnp.zeros((L,), jnp.int32),
             jnp.float32(NEG)))

        h_tgt.wait()

        def sum_body(b, acc):
            base = b * (L * 16)
            for u in range(16):
                acc = acc + tgt_v[pl.ds(base + u * L, L)]
            return acc

        acc = lax.fori_loop(0, N // (L * 16), sum_body,
                            jnp.zeros((L,), jnp.int32))
        tsum = jnp.sum(acc)

        cand_v, cand_i = _tie_fixup(cand_v, cand_i, fv_ref, fi_ref, iota)
        bits = plsc.load_gather(tgt_v, [cand_i]).astype(jnp.float32)
        hits = bits * mask12
        tp = plsc.cumsum(hits)
        metric = jnp.sum(tp * inv_ranks * hits)
        denom_i = jnp.minimum(jnp.int32(K), tsum)
        recip = jnp.max(jnp.where(iota == denom_i, rec_denom, NEG))
        ap = metric * recip

        res_v[k, :] = jnp.where(iota == 0, ap, 0.0)
        out_handles.append(
            pltpu.async_copy(res_v.at[k], out_hbm.at[row], sem_o))
        if k + 1 < ROWS_PER_W:
            h_tgt = pltpu.async_copy(targets_hbm.at[row + 1], tgt_v, sem_t)

    for h in out_handles:
        h.wait()


@jax.jit
def _sc_map(logits, targets):
    mesh = plsc.VectorSubcoreMesh(core_axis_name="c", subcore_axis_name="s")
    f = pl.kernel(
        _sc_body,
        out_type=jax.ShapeDtypeStruct((B, L), jnp.float32),
        mesh=mesh,
        scratch_types=[
            pltpu.VMEM((N,), jnp.float32),
            pltpu.VMEM((N,), jnp.float32),
            pltpu.VMEM((N,), jnp.int32),
            pltpu.VMEM((2, L), jnp.float32),
            pltpu.VMEM((ROWS_PER_W, L), jnp.float32),
            pltpu.VMEM((L,), jnp.float32),
            pltpu.VMEM((L,), jnp.int32),
            pltpu.SemaphoreType.DMA,
            pltpu.SemaphoreType.DMA,
            pltpu.SemaphoreType.DMA,
        ],
        compiler_params=pltpu.CompilerParams(needs_layout_passes=False),
    )
    inv_ranks = 1.0 / (jnp.arange(L, dtype=jnp.float32) + 1.0)
    rec = jnp.arange(L, dtype=jnp.float32)
    rec_denom = jnp.where((rec >= 1) & (rec <= K), 1.0 / jnp.maximum(rec, 1.0),
                          jnp.where(rec == 0, jnp.inf, 0.0))
    tab = jnp.stack([inv_ranks, rec_denom]).astype(jnp.float32)
    return f(logits, targets, tab)


def kernel(logits, targets):
    return jnp.sum(_sc_map(logits, targets))


# speedup vs baseline: 1.4601x; 1.0006x over previous
import jax
import jax.numpy as jnp
from jax import lax
from jax.experimental import pallas as pl
from jax.experimental.pallas import tpu as pltpu
from jax.experimental.pallas import tpu_sc as plsc

B = 128
N = 32768
K = 12
L = 16
NW = 32
ROWS_PER_W = B // NW
BLOCK = 1024
SUB = 128
NEG = -3.0e38


def _lane(x, k):
    i = lax.iota(jnp.int32, L)
    return jnp.max(jnp.where(i == k, x, NEG))


def _tree_max(vs):
    while len(vs) > 1:
        vs = [jnp.maximum(vs[i], vs[i + 1]) for i in range(0, len(vs) - 1, 2)] \
            + ([vs[-1]] if len(vs) % 2 else [])
    return vs[0]


def _merge16(av, ai, bv, bi):
    rbv = lax.rev(bv, (0,))
    rbi = lax.rev(bi, (0,))
    take = (av > rbv) | ((av == rbv) & (ai < rbi))
    nv = jnp.where(take, av, rbv)
    ni = jnp.where(take, ai, rbi)
    return plsc.sort_key_val(nv, ni, descending=True)


def _sub_top16(log_v, sbase):
    iota = lax.iota(jnp.int32, L)
    pairs = []
    for u in range(8):
        v = log_v[pl.ds(sbase + u * L, L)]
        idx = sbase + u * L + iota
        pairs.append(plsc.sort_key_val(v, idx, descending=True))
    while len(pairs) > 1:
        pairs = [_merge16(*pairs[i], *pairs[i + 1])
                 for i in range(0, len(pairs), 2)]
    return pairs[0]


def _tie_fixup(cand_v, cand_i, fv_ref, fi_ref, iota):
    for phase in range(2):
        if phase == 0:
            partner = jnp.bitwise_xor(iota, 1)
        else:
            partner = jnp.where((iota >= 1) & (iota <= 14),
                                jnp.bitwise_xor(iota - 1, 1) + 1, iota)
        fv_ref[...] = cand_v
        fi_ref[...] = cand_i
        pv = plsc.load_gather(fv_ref, [partner])
        pi = plsc.load_gather(fi_ref, [partner])
        win = (cand_v > pv) | ((cand_v == pv) & (cand_i < pi))
        lower = iota < partner
        keep = (lower & win) | (~lower & ~win)
        cand_v = jnp.where(keep, cand_v, pv)
        cand_i = jnp.where(keep, cand_i, pi)
    return cand_v, cand_i


def _sc_body(logits_hbm, targets_hbm, tab_hbm, out_hbm,
             log_a, log_b, tgt_v, tab_v, res_v, fv_ref, fi_ref,
             sem_l, sem_t, sem_o):
    wid = lax.axis_index("c") * 16 + lax.axis_index("s")
    iota = lax.iota(jnp.int32, L)
    pltpu.sync_copy(tab_hbm, tab_v)
    inv_ranks = tab_v[0]
    rec_denom = tab_v[1]
    mask12 = (iota < K).astype(jnp.float32)

    r0 = wid * ROWS_PER_W
    logbufs = [log_a, log_b]
    h_log = pltpu.async_copy(logits_hbm.at[r0], log_a, sem_l)
    h_tgt = pltpu.async_copy(targets_hbm.at[r0], tgt_v, sem_t)
    out_handles = []

    for k in range(ROWS_PER_W):
        row = r0 + k
        log_v = logbufs[k % 2]
        h_log.wait()
        if k + 1 < ROWS_PER_W:
            h_log = pltpu.async_copy(
                logits_hbm.at[row + 1], logbufs[(k + 1) % 2], sem_l)

        import contextlib
        scope = jax.named_scope
        def blk_body(b, carry, log_v=log_v):
            cand_v, cand_i, t = carry
            base = b * BLOCK
            accs = []
            for u in range(8):
                a = log_v[pl.ds(base + u * SUB, L)]
                for j in range(1, SUB // L):
                    a = jnp.maximum(a, log_v[pl.ds(base + u * SUB + j * L, L)])
                accs.append(a)
            bmax = jnp.max(_tree_max(accs))

            def slow(carry):
                cv, ci, tt = carry
                tvec = jnp.full((L,), tt)
                bits = jnp.zeros((L,), jnp.int32)
                for u in range(8):
                    cnt = plsc.all_reduce_population_count(accs[u] > tvec)
                    bits = bits | (jnp.minimum(cnt, 1) << u)
                bmap = jnp.max(bits)

                def sb_body(sb, carry):
                    def go(carry):
                        cv, ci, t_in = carry
                        sv, si = _sub_top16(log_v, base + sb * SUB)
                        cv2, ci2 = _merge16(sv, si, cv, ci)
                        return cv2, ci2, t_in
                    hit = (lax.shift_right_logical(bmap, sb) & 1) > 0
                    return lax.cond(hit, lambda: go(carry), lambda: carry)

                cv, ci, _ = lax.fori_loop(0, BLOCK // SUB, sb_body,
                                          (cv, ci, tt))
                return cv, ci, _lane(cv, K - 1)

            return lax.cond(bmax > t, lambda: slow(carry), lambda: carry)

        with scope("scan"):
            cand_v, cand_i, _ = lax.fori_loop(
                0, N // BLOCK, blk_body,
                (jnp.full((L,), NEG, jnp.float32), jnp.zeros((L,), jnp.int32),
                 jnp.float32(NEG)))

        h_tgt.wait()

        def sum_body(b, acc):
            base = b * (L * 16)
            for u in range(16):
                acc = acc + tgt_v[pl.ds(base + u * L, L)]
            return acc

        with scope("tsum"):
            acc = lax.fori_loop(0, N // (L * 16), sum_body,
                                jnp.zeros((L,), jnp.int32))
            tsum = jnp.sum(acc)

        cand_v, cand_i = _tie_fixup(cand_v, cand_i, fv_ref, fi_ref, iota)
        bits = plsc.load_gather(tgt_v, [cand_i]).astype(jnp.float32)
        hits = bits * mask12
        tp = plsc.cumsum(hits)
        metric = jnp.sum(tp * inv_ranks * hits)
        denom_i = jnp.minimum(jnp.int32(K), tsum)
        recip = jnp.max(jnp.where(iota == denom_i, rec_denom, NEG))
        ap = metric * recip

        res_v[k, :] = jnp.where(iota == 0, ap, 0.0)
        out_handles.append(
            pltpu.async_copy(res_v.at[k], out_hbm.at[row], sem_o))
        if k + 1 < ROWS_PER_W:
            h_tgt = pltpu.async_copy(targets_hbm.at[row + 1], tgt_v, sem_t)

    for h in out_handles:
        h.wait()


@jax.jit
def _sc_map(logits, targets):
    mesh = plsc.VectorSubcoreMesh(core_axis_name="c", subcore_axis_name="s")
    f = pl.kernel(
        _sc_body,
        out_type=jax.ShapeDtypeStruct((B, L), jnp.float32),
        mesh=mesh,
        scratch_types=[
            pltpu.VMEM((N,), jnp.float32),
            pltpu.VMEM((N,), jnp.float32),
            pltpu.VMEM((N,), jnp.int32),
            pltpu.VMEM((2, L), jnp.float32),
            pltpu.VMEM((ROWS_PER_W, L), jnp.float32),
            pltpu.VMEM((L,), jnp.float32),
            pltpu.VMEM((L,), jnp.int32),
            pltpu.SemaphoreType.DMA,
            pltpu.SemaphoreType.DMA,
            pltpu.SemaphoreType.DMA,
        ],
        compiler_params=pltpu.CompilerParams(needs_layout_passes=False),
    )
    inv_ranks = 1.0 / (jnp.arange(L, dtype=jnp.float32) + 1.0)
    rec = jnp.arange(L, dtype=jnp.float32)
    rec_denom = jnp.where((rec >= 1) & (rec <= K), 1.0 / jnp.maximum(rec, 1.0),
                          jnp.where(rec == 0, jnp.inf, 0.0))
    tab = jnp.stack([inv_ranks, rec_denom]).astype(jnp.float32)
    return f(logits, targets, tab)


def kernel(logits, targets):
    return jnp.sum(_sc_map(logits, targets))


# speedup vs baseline: 1.6865x; 1.1551x over previous
import jax
import jax.numpy as jnp
from jax import lax
from jax.experimental import pallas as pl
from jax.experimental.pallas import tpu as pltpu
from jax.experimental.pallas import tpu_sc as plsc

B = 128
N = 32768
K = 12
L = 16
NW = 32
ROWS_PER_W = B // NW
BLOCK = 1024
SUB = 128
NEG = -3.0e38


def _lane(x, k):
    i = lax.iota(jnp.int32, L)
    return jnp.max(jnp.where(i == k, x, NEG))


def _tree_max(vs):
    while len(vs) > 1:
        vs = [jnp.maximum(vs[i], vs[i + 1]) for i in range(0, len(vs) - 1, 2)] \
            + ([vs[-1]] if len(vs) % 2 else [])
    return vs[0]


def _merge16(av, ai, bv, bi):
    rbv = lax.rev(bv, (0,))
    rbi = lax.rev(bi, (0,))
    take = (av > rbv) | ((av == rbv) & (ai < rbi))
    nv = jnp.where(take, av, rbv)
    ni = jnp.where(take, ai, rbi)
    return plsc.sort_key_val(nv, ni, descending=True)


def _merge_vals(a, b):
    mx = jnp.maximum(a, lax.rev(b, (0,)))
    return plsc.sort_key_val(mx, lax.iota(jnp.int32, L), descending=True)[0]


def _sub_top16(log_v, sbase):
    iota = lax.iota(jnp.int32, L)
    pairs = []
    for u in range(8):
        v = log_v[pl.ds(sbase + u * L, L)]
        idx = sbase + u * L + iota
        pairs.append(plsc.sort_key_val(v, idx, descending=True))
    while len(pairs) > 1:
        pairs = [_merge16(*pairs[i], *pairs[i + 1])
                 for i in range(0, len(pairs), 2)]
    return pairs[0]


def _tie_fixup(cand_v, cand_i, fv_ref, fi_ref, iota):
    for phase in range(2):
        if phase == 0:
            partner = jnp.bitwise_xor(iota, 1)
        else:
            partner = jnp.where((iota >= 1) & (iota <= 14),
                                jnp.bitwise_xor(iota - 1, 1) + 1, iota)
        fv_ref[...] = cand_v
        fi_ref[...] = cand_i
        pv = plsc.load_gather(fv_ref, [partner])
        pi = plsc.load_gather(fi_ref, [partner])
        win = (cand_v > pv) | ((cand_v == pv) & (cand_i < pi))
        lower = iota < partner
        keep = (lower & win) | (~lower & ~win)
        cand_v = jnp.where(keep, cand_v, pv)
        cand_i = jnp.where(keep, cand_i, pi)
    return cand_v, cand_i


def _sc_body(logits_hbm, targets_hbm, tab_hbm, out_hbm,
             log_a, log_b, tgt_v, m_v, bm_v, tab_v, res_v, fv_ref, fi_ref,
             sem_l, sem_t, sem_o):
    wid = lax.axis_index("c") * 16 + lax.axis_index("s")
    iota = lax.iota(jnp.int32, L)
    pltpu.sync_copy(tab_hbm, tab_v)
    inv_ranks = tab_v[0]
    rec_denom = tab_v[1]
    mask12 = (iota < K).astype(jnp.float32)

    r0 = wid * ROWS_PER_W
    logbufs = [log_a, log_b]
    h_log = pltpu.async_copy(logits_hbm.at[r0], log_a, sem_l)
    h_tgt = pltpu.async_copy(targets_hbm.at[r0], tgt_v, sem_t)
    out_handles = []

    for k in range(ROWS_PER_W):
        row = r0 + k
        log_v = logbufs[k % 2]
        h_log.wait()
        if k + 1 < ROWS_PER_W:
            h_log = pltpu.async_copy(
                logits_hbm.at[row + 1], logbufs[(k + 1) % 2], sem_l)

        def pha(b, c, log_v=log_v):
            base = b * BLOCK
            subs = []
            for u in range(8):
                a = log_v[pl.ds(base + u * SUB, L)]
                for j in range(1, SUB // L):
                    a = jnp.maximum(a, log_v[pl.ds(base + u * SUB + j * L, L)])
                m_v[pl.ds((b * 8 + u) * L, L)] = a
                subs.append(a)
            bm_v[pl.ds(b * L, L)] = _tree_max(subs)
            return c

        lax.fori_loop(0, N // BLOCK, pha, jnp.int32(0))

        pairs = []
        for b2 in range(N // BLOCK):
            v = bm_v[pl.ds(b2 * L, L)]
            pairs.append(plsc.sort_key_val(v, iota, descending=True)[0])
        while len(pairs) > 1:
            pairs = [_merge_vals(pairs[i], pairs[i + 1])
                     for i in range(0, len(pairs), 2)]
        theta = _lane(pairs[0], K - 1)
        tvec = jnp.full((L,), theta)

        bits0 = jnp.zeros((L,), jnp.int32)
        bits1 = jnp.zeros((L,), jnp.int32)
        for b2 in range(N // BLOCK):
            v = bm_v[pl.ds(b2 * L, L)]
            one = jnp.minimum(
                plsc.all_reduce_population_count(v >= tvec), 1) << (b2 % 16)
            if b2 < 16:
                bits0 = bits0 | one
            else:
                bits1 = bits1 | one
        s0 = jnp.max(bits0)
        s1 = jnp.max(bits1)

        def phc(b, carry, log_v=log_v):
            bsel = jnp.where(b < 16, s0, s1)
            hit = (lax.shift_right_logical(bsel, b & 15) & 1) > 0

            def go(carry):
                mbits = jnp.zeros((L,), jnp.int32)
                for u in range(8):
                    mv = m_v[pl.ds((b * 8 + u) * L, L)]
                    mbits = mbits | (jnp.minimum(
                        plsc.all_reduce_population_count(mv >= tvec), 1) << u)
                bmap = jnp.max(mbits)

                def sbody(u, carry):
                    def go2(carry):
                        cv, ci = carry
                        sv, si = _sub_top16(log_v, b * BLOCK + u * SUB)
                        cv2, ci2 = _merge16(sv, si, cv, ci)
                        return cv2, ci2
                    hit2 = (lax.shift_right_logical(bmap, u) & 1) > 0
                    return lax.cond(hit2, lambda: go2(carry), lambda: carry)

                return lax.fori_loop(0, BLOCK // SUB, sbody, carry)

            return lax.cond(hit, lambda: go(carry), lambda: carry)

        cand_v, cand_i = lax.fori_loop(
            0, N // BLOCK, phc,
            (jnp.full((L,), NEG, jnp.float32), jnp.zeros((L,), jnp.int32)))

        h_tgt.wait()

        def sum_body(b, acc):
            base = b * (L * 16)
            for u in range(16):
                acc = acc + tgt_v[pl.ds(base + u * L, L)]
            return acc

        acc = lax.fori_loop(0, N // (L * 16), sum_body,
                            jnp.zeros((L,), jnp.int32))
        tsum = jnp.sum(acc)

        cand_v, cand_i = _tie_fixup(cand_v, cand_i, fv_ref, fi_ref, iota)
        bits = plsc.load_gather(tgt_v, [cand_i]).astype(jnp.float32)
        hits = bits * mask12
        tp = plsc.cumsum(hits)
        metric = jnp.sum(tp * inv_ranks * hits)
        denom_i = jnp.minimum(jnp.int32(K), tsum)
        recip = jnp.max(jnp.where(iota == denom_i, rec_denom, NEG))
        ap = metric * recip

        res_v[k, :] = jnp.where(iota == 0, ap, 0.0)
        out_handles.append(
            pltpu.async_copy(res_v.at[k], out_hbm.at[row], sem_o))
        if k + 1 < ROWS_PER_W:
            h_tgt = pltpu.async_copy(targets_hbm.at[row + 1], tgt_v, sem_t)

    for h in out_handles:
        h.wait()


@jax.jit
def _sc_map(logits, targets):
    mesh = plsc.VectorSubcoreMesh(core_axis_name="c", subcore_axis_name="s")
    f = pl.kernel(
        _sc_body,
        out_type=jax.ShapeDtypeStruct((B, L), jnp.float32),
        mesh=mesh,
        scratch_types=[
            pltpu.VMEM((N,), jnp.float32),
            pltpu.VMEM((N,), jnp.float32),
            pltpu.VMEM((N,), jnp.int32),
            pltpu.VMEM((N // 8,), jnp.float32),
            pltpu.VMEM((N // 64,), jnp.float32),
            pltpu.VMEM((2, L), jnp.float32),
            pltpu.VMEM((ROWS_PER_W, L), jnp.float32),
            pltpu.VMEM((L,), jnp.float32),
            pltpu.VMEM((L,), jnp.int32),
            pltpu.SemaphoreType.DMA,
            pltpu.SemaphoreType.DMA,
            pltpu.SemaphoreType.DMA,
        ],
        compiler_params=pltpu.CompilerParams(needs_layout_passes=False),
    )
    inv_ranks = 1.0 / (jnp.arange(L, dtype=jnp.float32) + 1.0)
    rec = jnp.arange(L, dtype=jnp.float32)
    rec_denom = jnp.where((rec >= 1) & (rec <= K), 1.0 / jnp.maximum(rec, 1.0),
                          jnp.where(rec == 0, jnp.inf, 0.0))
    tab = jnp.stack([inv_ranks, rec_denom]).astype(jnp.float32)
    return f(logits, targets, tab)


def kernel(logits, targets):
    return jnp.sum(_sc_map(logits, targets))
